# baseline (speedup 1.0000x reference)
import dataclasses
import functools

import jax
import jax.numpy as jnp
from jax import lax
from jax.experimental import pallas as pl
from jax.experimental.pallas import tpu as pltpu
from jax.experimental.pallas import tpu_sc as plsc

B = 16384
D = 128
V = 1000
VPAD = 1024
NC = 2
NS = 16
NW = NC * NS
BPW = B // NW
HPS = B // NS


@functools.cache
def _build_sc_gather_hist():
  sc_mesh = plsc.VectorSubcoreMesh(
      core_axis_name="c", subcore_axis_name="s", num_cores=NC, num_subcores=NS
  )

  @functools.partial(
      pl.kernel,
      out_type=(
          jax.ShapeDtypeStruct((B, D), jnp.float32),
          jax.ShapeDtypeStruct((B,), jnp.float32),
      ),
      mesh=sc_mesh,
      compiler_params=dataclasses.replace(
          pltpu.CompilerParams(), needs_layout_passes=False
      )
      if "needs_layout_passes" in pltpu.CompilerParams.__dataclass_fields__
      else pltpu.CompilerParams(),
      scratch_types=[
          pltpu.VMEM((4, 128), jnp.int32),
          pltpu.VMEM((8, 128), jnp.int32),
          pltpu.VMEM((NS * VPAD,), jnp.float32),
          pltpu.VMEM((VPAD,), jnp.float32),
          pltpu.VMEM((NS, VPAD), jnp.float32),
          pltpu.VMEM((BPW, D), jnp.float32),
          pltpu.VMEM((BPW,), jnp.float32),
          pltpu.VMEM_SHARED((NS, VPAD), jnp.float32),
          pltpu.VMEM_SHARED((V, D), jnp.float32),
          pltpu.SemaphoreType.DMA,
      ],
  )
  def sc_gather_hist(
      center_hbm,
      ys2d_hbm,
      g_hbm,
      w_hbm,
      idx_v,
      hlbl_v,
      lhist_v,
      hist_v,
      allh_v,
      rows_v,
      w_v,
      shared_h,
      shared_c,
      sem,
  ):
    cid = lax.axis_index("c")
    sid = lax.axis_index("s")
    wid = sid * NC + cid

    pltpu.sync_copy(ys2d_hbm.at[pl.ds(wid * 4, 4)], idx_v)

    @pl.when(sid == 0)
    def _():
      pltpu.sync_copy(center_hbm, shared_c)

    pltpu.sync_copy(ys2d_hbm.at[pl.ds(sid * 8, 8)], hlbl_v)

    @pl.loop(0, NS * VPAD, step=16)
    def _(i):
      lhist_v[pl.ds(i, 16)] = jnp.zeros((16,), jnp.float32)

    lane_off = lax.iota(jnp.int32, 16) * VPAD
    ones16 = jnp.full((16,), 1.0, jnp.float32)
    for r in range(8):

      @pl.loop(0, 128, step=16)
      def _(i):
        lblv = hlbl_v[r, pl.ds(i, 16)]
        plsc.addupdate_scatter(lhist_v, [lblv + lane_off], ones16)

    @pl.loop(0, VPAD, step=16)
    def _(i):
      acc = lhist_v[pl.ds(i, 16)]
      for q in range(1, NS):
        acc = acc + lhist_v[pl.ds(q * VPAD + i, 16)]
      hist_v[pl.ds(i, 16)] = acc

    pltpu.sync_copy(hist_v, shared_h.at[sid])
    plsc.subcore_barrier()

    for j in range(4):
      pltpu.async_copy(
          shared_c.at[idx_v.at[j]], rows_v.at[pl.ds(j * 128, 128)], sem
      )

    pltpu.sync_copy(shared_h, allh_v)

    @pl.loop(0, VPAD, step=16)
    def _(i):
      acc = allh_v[0, pl.ds(i, 16)]
      for q in range(1, NS):
        acc = acc + allh_v[q, pl.ds(i, 16)]
      hist_v[pl.ds(i, 16)] = acc

    for j in range(4):

      @pl.loop(0, 128, step=16)
      def _(i):
        lbl = idx_v[j, pl.ds(i, 16)]
        cnt = plsc.load_gather(hist_v, [lbl])
        w_v[pl.ds(j * 128 + i, 16)] = 1.0 / cnt

    for j in range(4):
      pltpu.make_async_copy(
          shared_c.at[idx_v.at[j]], rows_v.at[pl.ds(j * 128, 128)], sem
      ).wait()
    pltpu.sync_copy(rows_v, g_hbm.at[pl.ds(wid * BPW, BPW)])
    pltpu.sync_copy(w_v, w_hbm.at[pl.ds(wid * BPW, BPW)])

  return sc_gather_hist


_TC_BLK = 4096


_TC_ROWS = _TC_BLK // 128


def _tc_inv_body(xs_ref, out_ref):
  xs = xs_ref[...]
  s = jnp.sum(xs * xs, axis=2)
  out_ref[...] = 1.0 / jnp.maximum(jnp.sqrt(s), 1e-12)


def _tc_inv(xs3):
  return pl.pallas_call(
      _tc_inv_body,
      grid=(B // _TC_BLK,),
      in_specs=[pl.BlockSpec((_TC_ROWS, 128, D), lambda i: (i, 0, 0))],
      out_specs=pl.BlockSpec((_TC_ROWS, 128), lambda i: (i, 0)),
      out_shape=jax.ShapeDtypeStruct((128, 128), jnp.float32),
      compiler_params=pltpu.CompilerParams(dimension_semantics=("parallel",)),
  )(xs3)


def _tc_body(xs_ref, g_ref, w_ref, inv_ref, out_ref):
  t = xs_ref[...] * inv_ref[...][:, :, None] - g_ref[...]
  out_ref[0, 0, 0] = jnp.sum(w_ref[...][:, :, None] * t * t)


def _tc_reduce(xs3, g, w, inv2):
  return pl.pallas_call(
      _tc_body,
      grid=(B // _TC_BLK,),
      in_specs=[
          pl.BlockSpec((_TC_ROWS, 128, D), lambda i: (i, 0, 0)),
          pl.BlockSpec((_TC_ROWS, 128, D), lambda i: (i, 0, 0)),
          pl.BlockSpec((_TC_ROWS, 128), lambda i: (i, 0)),
          pl.BlockSpec((_TC_ROWS, 128), lambda i: (i, 0)),
      ],
      out_specs=pl.BlockSpec(
          (1, 1, 1), lambda i: (i, 0, 0), memory_space=pltpu.SMEM
      ),
      out_shape=jax.ShapeDtypeStruct((B // _TC_BLK, 1, 1), jnp.float32),
      compiler_params=pltpu.CompilerParams(dimension_semantics=("parallel",)),
  )(xs3, g.reshape(128, 128, D), w.reshape(128, 128), inv2)


@jax.jit
def kernel(xs, ys, center):
  ys2d = ys.astype(jnp.int32).reshape(128, 128)
  xs3 = xs.reshape(128, 128, D)
  g, w = _build_sc_gather_hist()(center, ys2d)
  inv2 = _tc_inv(xs3)
  partials = _tc_reduce(xs3, g, w, inv2)
  return jnp.sum(partials) / 2.0


# speedup vs baseline: 1.0750x; 1.0750x over previous
import dataclasses
import functools

import jax
import jax.numpy as jnp
from jax import lax
from jax.experimental import pallas as pl
from jax.experimental.pallas import tpu as pltpu
from jax.experimental.pallas import tpu_sc as plsc

B = 16384
D = 128
V = 1000
VPAD = 1024
NC = 2
NS = 16
NW = NC * NS
BPW = B // NW
HPS = B // NS


@functools.cache
def _build_sc_gather_hist():
  sc_mesh = plsc.VectorSubcoreMesh(
      core_axis_name="c", subcore_axis_name="s", num_cores=NC, num_subcores=NS
  )

  @functools.partial(
      pl.kernel,
      out_type=(
          jax.ShapeDtypeStruct((B, D), jnp.float32),
          jax.ShapeDtypeStruct((B,), jnp.float32),
      ),
      mesh=sc_mesh,
      compiler_params=dataclasses.replace(
          pltpu.CompilerParams(), needs_layout_passes=False
      )
      if "needs_layout_passes" in pltpu.CompilerParams.__dataclass_fields__
      else pltpu.CompilerParams(),
      scratch_types=[
          pltpu.VMEM((4, 128), jnp.int32),
          pltpu.VMEM((8, 128), jnp.int32),
          pltpu.VMEM((NS * VPAD,), jnp.float32),
          pltpu.VMEM((VPAD,), jnp.float32),
          pltpu.VMEM((NS, VPAD), jnp.float32),
          pltpu.VMEM((BPW, D), jnp.float32),
          pltpu.VMEM((BPW,), jnp.float32),
          pltpu.VMEM_SHARED((NS, VPAD), jnp.float32),
          pltpu.SemaphoreType.DMA,
      ],
  )
  def sc_gather_hist(
      center_hbm,
      ys2d_hbm,
      g_hbm,
      w_hbm,
      idx_v,
      hlbl_v,
      lhist_v,
      hist_v,
      allh_v,
      rows_v,
      w_v,
      shared_h,
      sem,
  ):
    cid = lax.axis_index("c")
    sid = lax.axis_index("s")
    wid = sid * NC + cid

    pltpu.sync_copy(ys2d_hbm.at[pl.ds(wid * 4, 4)], idx_v)

    for j in range(4):
      pltpu.async_copy(
          center_hbm.at[idx_v.at[j]], rows_v.at[pl.ds(j * 128, 128)], sem
      )

    pltpu.sync_copy(ys2d_hbm.at[pl.ds(sid * 8, 8)], hlbl_v)

    @pl.loop(0, NS * VPAD, step=16)
    def _(i):
      lhist_v[pl.ds(i, 16)] = jnp.zeros((16,), jnp.float32)

    lane_off = lax.iota(jnp.int32, 16) * VPAD
    ones16 = jnp.full((16,), 1.0, jnp.float32)
    for r in range(8):

      @pl.loop(0, 128, step=16)
      def _(i):
        lblv = hlbl_v[r, pl.ds(i, 16)]
        plsc.addupdate_scatter(lhist_v, [lblv + lane_off], ones16)

    @pl.loop(0, VPAD, step=16)
    def _(i):
      acc = lhist_v[pl.ds(i, 16)]
      for q in range(1, NS):
        acc = acc + lhist_v[pl.ds(q * VPAD + i, 16)]
      hist_v[pl.ds(i, 16)] = acc

    pltpu.sync_copy(hist_v, shared_h.at[sid])
    plsc.subcore_barrier()
    pltpu.sync_copy(shared_h, allh_v)

    @pl.loop(0, VPAD, step=16)
    def _(i):
      acc = allh_v[0, pl.ds(i, 16)]
      for q in range(1, NS):
        acc = acc + allh_v[q, pl.ds(i, 16)]
      hist_v[pl.ds(i, 16)] = acc

    for j in range(4):

      @pl.loop(0, 128, step=16)
      def _(i):
        lbl = idx_v[j, pl.ds(i, 16)]
        cnt = plsc.load_gather(hist_v, [lbl])
        w_v[pl.ds(j * 128 + i, 16)] = 1.0 / cnt

    for j in range(4):
      pltpu.make_async_copy(
          center_hbm.at[idx_v.at[j]], rows_v.at[pl.ds(j * 128, 128)], sem
      ).wait()
    pltpu.sync_copy(rows_v, g_hbm.at[pl.ds(wid * BPW, BPW)])
    pltpu.sync_copy(w_v, w_hbm.at[pl.ds(wid * BPW, BPW)])

  return sc_gather_hist


_TC_BLK = 4096


_TC_ROWS = _TC_BLK // 128


def _tc_norm_body(xs_ref, out_ref):
  xs = xs_ref[...]
  s = jnp.sum(xs * xs, axis=2, keepdims=True)
  inv = 1.0 / jnp.maximum(jnp.sqrt(s), 1e-12)
  out_ref[...] = (xs * inv).astype(jnp.bfloat16)


def _tc_norm(xs3):
  return pl.pallas_call(
      _tc_norm_body,
      grid=(B // _TC_BLK,),
      in_specs=[pl.BlockSpec((_TC_ROWS, 128, D), lambda i: (i, 0, 0))],
      out_specs=pl.BlockSpec((_TC_ROWS, 128, D), lambda i: (i, 0, 0)),
      out_shape=jax.ShapeDtypeStruct((128, 128, D), jnp.bfloat16),
      compiler_params=pltpu.CompilerParams(dimension_semantics=("parallel",)),
  )(xs3)


def _tc_body(xh_ref, g_ref, w_ref, out_ref):
  t = xh_ref[...].astype(jnp.float32) - g_ref[...]
  out_ref[0, 0, 0] = jnp.sum(w_ref[...][:, :, None] * t * t)


def _tc_reduce(xh3, g, w):
  return pl.pallas_call(
      _tc_body,
      grid=(B // _TC_BLK,),
      in_specs=[
          pl.BlockSpec((_TC_ROWS, 128, D), lambda i: (i, 0, 0)),
          pl.BlockSpec((_TC_ROWS, 128, D), lambda i: (i, 0, 0)),
          pl.BlockSpec((_TC_ROWS, 128), lambda i: (i, 0)),
      ],
      out_specs=pl.BlockSpec(
          (1, 1, 1), lambda i: (i, 0, 0), memory_space=pltpu.SMEM
      ),
      out_shape=jax.ShapeDtypeStruct((B // _TC_BLK, 1, 1), jnp.float32),
      compiler_params=pltpu.CompilerParams(dimension_semantics=("parallel",)),
  )(xh3, g.reshape(128, 128, D), w.reshape(128, 128))


@jax.jit
def kernel(xs, ys, center):
  ys2d = ys.astype(jnp.int32).reshape(128, 128)
  xs3 = xs.reshape(128, 128, D)
  g, w = _build_sc_gather_hist()(center, ys2d)
  xh3 = _tc_norm(xs3)
  partials = _tc_reduce(xh3, g, w)
  return jnp.sum(partials) / 2.0


# speedup vs baseline: 1.1327x; 1.0537x over previous
import dataclasses
import functools

import jax
import jax.numpy as jnp
from jax import lax
from jax.experimental import pallas as pl
from jax.experimental.pallas import tpu as pltpu
from jax.experimental.pallas import tpu_sc as plsc

B = 16384
D = 128
V = 1000
VPAD = 1024
NC = 2
NS = 16
NW = NC * NS
BPW = B // NW
HPS = B // NS


@functools.cache
def _build_sc_gather_hist():
  sc_mesh = plsc.VectorSubcoreMesh(
      core_axis_name="c", subcore_axis_name="s", num_cores=NC, num_subcores=NS
  )

  @functools.partial(
      pl.kernel,
      out_type=(
          jax.ShapeDtypeStruct((B, D), jnp.float32),
          jax.ShapeDtypeStruct((B,), jnp.float32),
      ),
      mesh=sc_mesh,
      compiler_params=dataclasses.replace(
          pltpu.CompilerParams(), needs_layout_passes=False
      )
      if "needs_layout_passes" in pltpu.CompilerParams.__dataclass_fields__
      else pltpu.CompilerParams(),
      scratch_types=[
          pltpu.VMEM((4, 128), jnp.int32),
          pltpu.VMEM((8, 128), jnp.int32),
          pltpu.VMEM((NS * VPAD,), jnp.float32),
          pltpu.VMEM((VPAD,), jnp.float32),
          pltpu.VMEM((NS, VPAD), jnp.float32),
          pltpu.VMEM((BPW, D), jnp.float32),
          pltpu.VMEM((BPW,), jnp.float32),
          pltpu.VMEM_SHARED((NS, VPAD), jnp.float32),
          pltpu.SemaphoreType.DMA,
          pltpu.SemaphoreType.DMA,
      ],
  )
  def sc_gather_hist(
      center_hbm,
      ys2d_hbm,
      g_hbm,
      w_hbm,
      idx_v,
      hlbl_v,
      lhist_v,
      hist_v,
      allh_v,
      rows_v,
      w_v,
      shared_h,
      sem,
      wsem,
  ):
    cid = lax.axis_index("c")
    sid = lax.axis_index("s")
    wid = sid * NC + cid

    pltpu.sync_copy(ys2d_hbm.at[pl.ds(wid * 4, 4)], idx_v)

    for j in range(4):
      pltpu.async_copy(
          center_hbm.at[idx_v.at[j]], rows_v.at[pl.ds(j * 128, 128)], sem
      )

    pltpu.sync_copy(ys2d_hbm.at[pl.ds(sid * 8, 8)], hlbl_v)

    @pl.loop(0, NS * VPAD, step=16)
    def _(i):
      lhist_v[pl.ds(i, 16)] = jnp.zeros((16,), jnp.float32)

    lane_off = lax.iota(jnp.int32, 16) * VPAD
    ones16 = jnp.full((16,), 1.0, jnp.float32)
    for r in range(8):

      @pl.loop(0, 128, step=16)
      def _(i):
        lblv = hlbl_v[r, pl.ds(i, 16)]
        plsc.addupdate_scatter(lhist_v, [lblv + lane_off], ones16)

    @pl.loop(0, VPAD, step=16)
    def _(i):
      acc = lhist_v[pl.ds(i, 16)]
      for q in range(1, NS):
        acc = acc + lhist_v[pl.ds(q * VPAD + i, 16)]
      hist_v[pl.ds(i, 16)] = acc

    pltpu.sync_copy(hist_v, shared_h.at[sid])
    plsc.subcore_barrier()

    for j in range(4):
      pltpu.make_async_copy(
          center_hbm.at[idx_v.at[j]], rows_v.at[pl.ds(j * 128, 128)], sem
      ).wait()
      pltpu.async_copy(
          rows_v.at[pl.ds(j * 128, 128)],
          g_hbm.at[pl.ds(wid * BPW + j * 128, 128)],
          wsem,
      )

    pltpu.sync_copy(shared_h, allh_v)

    @pl.loop(0, VPAD, step=16)
    def _(i):
      acc = allh_v[0, pl.ds(i, 16)]
      for q in range(1, NS):
        acc = acc + allh_v[q, pl.ds(i, 16)]
      hist_v[pl.ds(i, 16)] = acc

    for j in range(4):

      @pl.loop(0, 128, step=16)
      def _(i):
        lbl = idx_v[j, pl.ds(i, 16)]
        cnt = plsc.load_gather(hist_v, [lbl])
        w_v[pl.ds(j * 128 + i, 16)] = 1.0 / cnt

    pltpu.sync_copy(w_v, w_hbm.at[pl.ds(wid * BPW, BPW)])
    for j in range(4):
      pltpu.make_async_copy(
          rows_v.at[pl.ds(j * 128, 128)],
          g_hbm.at[pl.ds(wid * BPW + j * 128, 128)],
          wsem,
      ).wait()

  return sc_gather_hist


_TC_BLK = 4096


_TC_ROWS = _TC_BLK // 128


def _tc_norm_body(xs_ref, out_ref):
  xs = xs_ref[...]
  s = jnp.sum(xs * xs, axis=2, keepdims=True)
  inv = 1.0 / jnp.maximum(jnp.sqrt(s), 1e-12)
  out_ref[...] = (xs * inv).astype(jnp.bfloat16)


def _tc_norm(xs3):
  return pl.pallas_call(
      _tc_norm_body,
      grid=(B // _TC_BLK,),
      in_specs=[pl.BlockSpec((_TC_ROWS, 128, D), lambda i: (i, 0, 0))],
      out_specs=pl.BlockSpec((_TC_ROWS, 128, D), lambda i: (i, 0, 0)),
      out_shape=jax.ShapeDtypeStruct((128, 128, D), jnp.bfloat16),
      compiler_params=pltpu.CompilerParams(dimension_semantics=("parallel",)),
  )(xs3)


def _tc_body(xh_ref, g_ref, w_ref, out_ref):
  t = xh_ref[...].astype(jnp.float32) - g_ref[...]
  out_ref[0, 0, 0] = jnp.sum(w_ref[...][:, :, None] * t * t)


def _tc_reduce(xh3, g, w):
  return pl.pallas_call(
      _tc_body,
      grid=(B // _TC_BLK,),
      in_specs=[
          pl.BlockSpec((_TC_ROWS, 128, D), lambda i: (i, 0, 0)),
          pl.BlockSpec((_TC_ROWS, 128, D), lambda i: (i, 0, 0)),
          pl.BlockSpec((_TC_ROWS, 128), lambda i: (i, 0)),
      ],
      out_specs=pl.BlockSpec(
          (1, 1, 1), lambda i: (i, 0, 0), memory_space=pltpu.SMEM
      ),
      out_shape=jax.ShapeDtypeStruct((B // _TC_BLK, 1, 1), jnp.float32),
      compiler_params=pltpu.CompilerParams(dimension_semantics=("parallel",)),
  )(xh3, g.reshape(128, 128, D), w.reshape(128, 128))


@jax.jit
def kernel(xs, ys, center):
  ys2d = ys.astype(jnp.int32).reshape(128, 128)
  xs3 = xs.reshape(128, 128, D)
  g, w = _build_sc_gather_hist()(center, ys2d)
  xh3 = _tc_norm(xs3)
  partials = _tc_reduce(xh3, g, w)
  return jnp.sum(partials) / 2.0


# speedup vs baseline: 1.1347x; 1.0018x over previous
import dataclasses
import functools

import jax
import jax.numpy as jnp
from jax import lax
from jax.experimental import pallas as pl
from jax.experimental.pallas import tpu as pltpu
from jax.experimental.pallas import tpu_sc as plsc

B = 16384
D = 128
V = 1000
VPAD = 1024
NC = 2
NS = 16
NW = NC * NS
BPW = B // NW
HPS = B // NS


@functools.cache
def _build_sc_gather_hist():
  sc_mesh = plsc.VectorSubcoreMesh(
      core_axis_name="c", subcore_axis_name="s", num_cores=NC, num_subcores=NS
  )

  @functools.partial(
      pl.kernel,
      out_type=(
          jax.ShapeDtypeStruct((B, D), jnp.float32),
          jax.ShapeDtypeStruct((B,), jnp.float32),
      ),
      mesh=sc_mesh,
      compiler_params=dataclasses.replace(
          pltpu.CompilerParams(), needs_layout_passes=False
      )
      if "needs_layout_passes" in pltpu.CompilerParams.__dataclass_fields__
      else pltpu.CompilerParams(),
      scratch_types=[
          pltpu.VMEM((4, 128), jnp.int32),
          pltpu.VMEM((8, 128), jnp.int32),
          pltpu.VMEM((NS * VPAD,), jnp.float32),
          pltpu.VMEM((VPAD,), jnp.float32),
          pltpu.VMEM((NS, VPAD), jnp.float32),
          pltpu.VMEM((BPW, D), jnp.float32),
          pltpu.VMEM((BPW,), jnp.float32),
          pltpu.VMEM_SHARED((NS, VPAD), jnp.float32),
          pltpu.SemaphoreType.DMA,
          pltpu.SemaphoreType.DMA,
      ],
  )
  def sc_gather_hist(
      center_hbm,
      ys2d_hbm,
      g_hbm,
      w_hbm,
      idx_v,
      hlbl_v,
      lhist_v,
      hist_v,
      allh_v,
      rows_v,
      w_v,
      shared_h,
      sem,
      wsem,
  ):
    cid = lax.axis_index("c")
    sid = lax.axis_index("s")
    wid = sid * NC + cid

    pltpu.sync_copy(ys2d_hbm.at[pl.ds(wid * 4, 4)], idx_v)

    for j in range(4):
      pltpu.async_copy(
          center_hbm.at[idx_v.at[j]], rows_v.at[pl.ds(j * 128, 128)], sem
      )

    pltpu.sync_copy(ys2d_hbm.at[pl.ds(sid * 8, 8)], hlbl_v)

    @pl.loop(0, NS * VPAD, step=16)
    def _(i):
      lhist_v[pl.ds(i, 16)] = jnp.zeros((16,), jnp.float32)

    lane_off = lax.iota(jnp.int32, 16) * VPAD
    ones16 = jnp.full((16,), 1.0, jnp.float32)
    for r in range(8):

      @pl.loop(0, 128, step=16)
      def _(i):
        lblv = hlbl_v[r, pl.ds(i, 16)]
        plsc.addupdate_scatter(lhist_v, [lblv + lane_off], ones16)

    @pl.loop(0, VPAD, step=16)
    def _(i):
      acc = lhist_v[pl.ds(i, 16)]
      for q in range(1, NS):
        acc = acc + lhist_v[pl.ds(q * VPAD + i, 16)]
      hist_v[pl.ds(i, 16)] = acc

    for j in range(4):
      pltpu.make_async_copy(
          center_hbm.at[idx_v.at[j]], rows_v.at[pl.ds(j * 128, 128)], sem
      ).wait()
      pltpu.async_copy(
          rows_v.at[pl.ds(j * 128, 128)],
          g_hbm.at[pl.ds(wid * BPW + j * 128, 128)],
          wsem,
      )

    pltpu.sync_copy(hist_v, shared_h.at[sid])
    plsc.subcore_barrier()
    pltpu.sync_copy(shared_h, allh_v)

    @pl.loop(0, VPAD, step=16)
    def _(i):
      acc = allh_v[0, pl.ds(i, 16)]
      for q in range(1, NS):
        acc = acc + allh_v[q, pl.ds(i, 16)]
      hist_v[pl.ds(i, 16)] = acc

    for j in range(4):

      @pl.loop(0, 128, step=16)
      def _(i):
        lbl = idx_v[j, pl.ds(i, 16)]
        cnt = plsc.load_gather(hist_v, [lbl])
        w_v[pl.ds(j * 128 + i, 16)] = 1.0 / cnt

    pltpu.sync_copy(w_v, w_hbm.at[pl.ds(wid * BPW, BPW)])
    for j in range(4):
      pltpu.make_async_copy(
          rows_v.at[pl.ds(j * 128, 128)],
          g_hbm.at[pl.ds(wid * BPW + j * 128, 128)],
          wsem,
      ).wait()

  return sc_gather_hist


_TC_BLK = 4096


_TC_ROWS = _TC_BLK // 128


def _tc_norm_body(xs_ref, out_ref):
  xs = xs_ref[...]
  s = jnp.sum(xs * xs, axis=2, keepdims=True)
  inv = 1.0 / jnp.maximum(jnp.sqrt(s), 1e-12)
  out_ref[...] = (xs * inv).astype(jnp.bfloat16)


def _tc_norm(xs3):
  return pl.pallas_call(
      _tc_norm_body,
      grid=(B // _TC_BLK,),
      in_specs=[pl.BlockSpec((_TC_ROWS, 128, D), lambda i: (i, 0, 0))],
      out_specs=pl.BlockSpec((_TC_ROWS, 128, D), lambda i: (i, 0, 0)),
      out_shape=jax.ShapeDtypeStruct((128, 128, D), jnp.bfloat16),
      compiler_params=pltpu.CompilerParams(dimension_semantics=("parallel",)),
  )(xs3)


def _tc_body(xh_ref, g_ref, w_ref, out_ref):
  t = xh_ref[...].astype(jnp.float32) - g_ref[...]
  out_ref[0, 0, 0] = jnp.sum(w_ref[...][:, :, None] * t * t)


def _tc_reduce(xh3, g, w):
  return pl.pallas_call(
      _tc_body,
      grid=(B // _TC_BLK,),
      in_specs=[
          pl.BlockSpec((_TC_ROWS, 128, D), lambda i: (i, 0, 0)),
          pl.BlockSpec((_TC_ROWS, 128, D), lambda i: (i, 0, 0)),
          pl.BlockSpec((_TC_ROWS, 128), lambda i: (i, 0)),
      ],
      out_specs=pl.BlockSpec(
          (1, 1, 1), lambda i: (i, 0, 0), memory_space=pltpu.SMEM
      ),
      out_shape=jax.ShapeDtypeStruct((B // _TC_BLK, 1, 1), jnp.float32),
      compiler_params=pltpu.CompilerParams(dimension_semantics=("parallel",)),
  )(xh3, g.reshape(128, 128, D), w.reshape(128, 128))


@jax.jit
def kernel(xs, ys, center):
  ys2d = ys.astype(jnp.int32).reshape(128, 128)
  xs3 = xs.reshape(128, 128, D)
  g, w = _build_sc_gather_hist()(center, ys2d)
  xh3 = _tc_norm(xs3)
  partials = _tc_reduce(xh3, g, w)
  return jnp.sum(partials) / 2.0
